# baseline (speedup 1.0000x reference)
import functools

import jax
import jax.numpy as jnp
from jax.experimental import pallas as pl
from jax.experimental.pallas import tpu as pltpu

_CHUNK = 512


def _router_kernel(x_hbm, w_ref, o_ref, buf, sems):
    rows = x_hbm.shape[0]
    half = rows // 2
    n = half // _CHUNK
    w = w_ref[...]

    def _copy(stream, i, slot):
        base = stream * half + i * _CHUNK
        return pltpu.make_async_copy(
            x_hbm.at[pl.ds(base, _CHUNK), :],
            buf.at[2 * stream + slot],
            sems.at[2 * stream + slot],
        )

    for s in range(2):
        _copy(0, s, s).start(priority=0)
        _copy(1, s, s).start(priority=1)

    def _softmax_dot(x_blk):
        logits = jax.lax.dot_general(
            x_blk, w,
            dimension_numbers=(((1,), (1,)), ((), ())),
            preferred_element_type=jnp.float32,
        )
        m = jnp.max(logits, axis=-1, keepdims=True)
        e = jnp.exp(logits - m)
        return e / jnp.sum(e, axis=-1, keepdims=True)

    def step(i, carry):
        slot = jax.lax.rem(i, 2)
        for stream in range(2):
            _copy(stream, i, slot).wait()
            base = stream * half + i * _CHUNK
            o_ref[pl.ds(base, _CHUNK), :] = _softmax_dot(buf[2 * stream + slot])

            @pl.when(i + 2 < n)
            def _():
                _copy(stream, i + 2, slot).start(priority=stream)

        return carry

    jax.lax.fori_loop(0, n, step, 0)


@functools.partial(jax.jit, static_argnames=("interpret",))
def kernel(x, expert_embeddings, interpret=False):
    B, S, H = x.shape
    E = expert_embeddings.shape[0]
    rows = B * S
    x2 = x.reshape(rows, H)
    out = pl.pallas_call(
        _router_kernel,
        in_specs=[
            pl.BlockSpec(memory_space=pltpu.MemorySpace.HBM),
            pl.BlockSpec((E, H), lambda: (0, 0)),
        ],
        out_specs=pl.BlockSpec((rows, E), lambda: (0, 0)),
        out_shape=jax.ShapeDtypeStruct((rows, E), jnp.float32),
        scratch_shapes=[
            pltpu.VMEM((4, _CHUNK, H), jnp.float32),
            pltpu.SemaphoreType.DMA((4,)),
        ],
        interpret=interpret,
    )(x2, expert_embeddings)
    return out.reshape(B, S, E)


# speedup vs baseline: 1.0779x; 1.0779x over previous
import functools

import jax
import jax.numpy as jnp
from jax.experimental import pallas as pl
from jax.experimental.pallas import tpu as pltpu

_TILE_S = 128


def _router_kernel(x_ref, w_ref, o_ref):
    b, t, h = x_ref.shape
    e = w_ref.shape[0]
    logits = jax.lax.dot_general(
        x_ref[...].reshape(b * t, h), w_ref[...],
        dimension_numbers=(((1,), (1,)), ((), ())),
        preferred_element_type=jnp.float32,
    )
    m = jnp.max(logits, axis=-1, keepdims=True)
    ex = jnp.exp(logits - m)
    probs = ex / jnp.sum(ex, axis=-1, keepdims=True)
    o_ref[...] = probs.reshape(b, t, e)


@functools.partial(jax.jit, static_argnames=("interpret",))
def kernel(x, expert_embeddings, interpret=False):
    B, S, H = x.shape
    E = expert_embeddings.shape[0]
    out = pl.pallas_call(
        _router_kernel,
        grid=(S // _TILE_S,),
        in_specs=[
            pl.BlockSpec((B, _TILE_S, H), lambda i: (0, i, 0)),
            pl.BlockSpec((E, H), lambda i: (0, 0)),
        ],
        out_specs=pl.BlockSpec((B, _TILE_S, E), lambda i: (0, i, 0)),
        out_shape=jax.ShapeDtypeStruct((B, S, E), jnp.float32),
        compiler_params=pltpu.CompilerParams(
            dimension_semantics=("arbitrary",),
        ),
        interpret=interpret,
    )(x, expert_embeddings)
    return out
